# baseline (speedup 1.0000x reference)
import functools

import jax
import jax.numpy as jnp
from jax import lax
from jax.experimental import pallas as pl
from jax.experimental.pallas import tpu as pltpu
from jax.experimental.pallas import tpu_sc as plsc

VOCAB = 1000000
EMBED_DIM = 32
SEQ = 50
BATCH = 16384

NC = 2
NS = 16
NW = NC * NS

LANE = 16
NBLK = 128
JB = 4
GROUP = JB * NBLK
J_PER_S = BATCH // NBLK
N_GROUPS_TOTAL = SEQ * BATCH // GROUP
G_PER_W = N_GROUPS_TOTAL // NW
IDX_PER_W = G_PER_W * GROUP
ROW_W = 8 * NBLK
DEPTH = 3


def _transpose_group(rows, tbuf):
    iota = lax.iota(jnp.int32, LANE)

    def dbody(d2, carry):
        for dd in range(2):
            d = d2 * 2 + dd
            dvec = jnp.full((LANE,), 0, jnp.int32) + d
            col0 = lax.rem(d, 8) * NBLK
            trow0 = (d // 8) * JB
            for jb in range(JB):
                for l0 in range(0, NBLK, LANE):
                    v = plsc.load_gather(rows, [iota + (jb * NBLK + l0), dvec])
                    tbuf[trow0 + jb, pl.ds(col0 + l0, LANE)] = v
        return carry

    lax.fori_loop(0, EMBED_DIM // 2, dbody, 0)


def _gather_body(ids_hbm, table_hbm, out_hbm, idx_v, rows, tb, sg, sw):
    wid = lax.axis_index("s") * NC + lax.axis_index("c")
    g0 = wid * G_PER_W

    pltpu.sync_copy(ids_hbm.at[pl.ds(g0 * GROUP, IDX_PER_W)], idx_v)

    def idx_slice(t):
        return idx_v.at[pl.ds(t * GROUP, GROUP)]

    def start_gather(t, u):
        pltpu.async_copy(table_hbm.at[idx_slice(t)], rows[u], sg[u])

    def wait_gather(t, u):
        pltpu.make_async_copy(table_hbm.at[idx_slice(t)], rows[u], sg[u]).wait()

    def start_write(t, u):
        c = (g0 + t) * JB
        s = c // J_PER_S
        j = lax.rem(c, J_PER_S)
        for i in range(EMBED_DIM // 8):
            pltpu.async_copy(tb[u].at[pl.ds(i * JB, JB)],
                             out_hbm.at[pl.ds((s * 4 + i) * J_PER_S + j, JB)],
                             sw[u])

    def wait_write(u):
        for _ in range(EMBED_DIM // 8):
            pltpu.make_async_copy(tb[u].at[pl.ds(0, JB)],
                                  out_hbm.at[pl.ds(0, JB)], sw[u]).wait()

    for t in range(DEPTH - 1):
        start_gather(t, t % DEPTH)

    def step_body(q, carry):
        for u in range(DEPTH):
            t = DEPTH * q + u

            @pl.when(t < G_PER_W)
            def _():
                @pl.when(t >= DEPTH)
                def _():
                    wait_write(u)
                wait_gather(t, u)
                _transpose_group(rows[u], tb[u])
                start_write(t, u)

            @pl.when(t + DEPTH - 1 < G_PER_W)
            def _():
                start_gather(t + DEPTH - 1, (u + DEPTH - 1) % DEPTH)
        return carry

    lax.fori_loop(0, (G_PER_W + DEPTH - 1) // DEPTH, step_body, 0)
    for u in range(DEPTH):
        wait_write(u)


@jax.jit
def _sc_gather(ids_flat, weight):
    mesh = plsc.VectorSubcoreMesh(core_axis_name="c", subcore_axis_name="s")

    def body(ids_hbm, table_hbm, out_hbm,
             idx_v, r0, r1, r2, t0, t1, t2, g0, g1, g2, w0, w1, w2):
        _gather_body(ids_hbm, table_hbm, out_hbm, idx_v,
                     [r0, r1, r2], [t0, t1, t2],
                     [g0, g1, g2], [w0, w1, w2])

    return pl.kernel(
        body,
        out_type=jax.ShapeDtypeStruct(
            (SEQ * (EMBED_DIM // 8) * J_PER_S, ROW_W), jnp.float32),
        mesh=mesh,
        scratch_types=(
            [pltpu.VMEM((IDX_PER_W,), jnp.int32)]
            + [pltpu.VMEM((GROUP, EMBED_DIM), jnp.float32)] * DEPTH
            + [pltpu.VMEM(((EMBED_DIM // 8) * JB, ROW_W), jnp.float32)] * DEPTH
            + [pltpu.SemaphoreType.DMA] * (2 * DEPTH)
        ),
        compiler_params=pltpu.CompilerParams(
            use_tc_tiling_on_sc=False, needs_layout_passes=False),
    )(ids_flat, weight)


def kernel(input_ids, weight):
    ids_flat = input_ids.astype(jnp.int32).T.reshape(-1)
    out2 = _sc_gather(ids_flat, weight)
    out5 = out2.reshape(SEQ, EMBED_DIM // 8, J_PER_S, 8, NBLK)
    return jnp.transpose(out5, (2, 4, 0, 1, 3)).reshape(BATCH, SEQ, EMBED_DIM)


# speedup vs baseline: 1.1116x; 1.1116x over previous
import functools

import jax
import jax.numpy as jnp
from jax import lax
from jax.experimental import pallas as pl
from jax.experimental.pallas import tpu as pltpu
from jax.experimental.pallas import tpu_sc as plsc

VOCAB = 1000000
EMBED_DIM = 32
SEQ = 50
BATCH = 16384

NC = 2
NS = 16
NW = NC * NS

LANE = 16
NBLK = 128
N_CHUNKS_TOTAL = SEQ * (BATCH // NBLK)
CH_PER_W = N_CHUNKS_TOTAL // NW
IDX_PER_W = CH_PER_W * NBLK
J_PER_S = BATCH // NBLK
ROW_W = 8 * NBLK
DEPTH = 4


def _transpose_chunk(rows, tbuf):
    iota128 = lax.iota(jnp.int32, LANE) * NBLK
    for l in range(NBLK):
        for d0 in range(0, EMBED_DIM, LANE):
            v = rows[l, pl.ds(d0, LANE)]
            plsc.store_scatter(tbuf, [iota128 + (d0 * NBLK + l)], v)


def _gather_body(ids_hbm, table_hbm, out_hbm, idx_v, rows, tb, sg, sw):
    wid = lax.axis_index("s") * NC + lax.axis_index("c")
    c0 = wid * CH_PER_W

    pltpu.sync_copy(ids_hbm.at[pl.ds(c0 * NBLK, IDX_PER_W)], idx_v)

    def idx_slice(t):
        return idx_v.at[pl.ds(t * NBLK, NBLK)]

    def start_gather(t, u):
        pltpu.async_copy(table_hbm.at[idx_slice(t)], rows[u], sg[u])

    def wait_gather(t, u):
        pltpu.make_async_copy(table_hbm.at[idx_slice(t)], rows[u], sg[u]).wait()

    def start_write(t, u):
        c = c0 + t
        s = c // J_PER_S
        j = lax.rem(c, J_PER_S)
        for i in range(EMBED_DIM // 8):
            pltpu.async_copy(tb[u].at[pl.ds(i * ROW_W, ROW_W)],
                             out_hbm.at[(s * 4 + i) * J_PER_S + j], sw[u])

    def wait_write(u):
        for _ in range(EMBED_DIM // 8):
            pltpu.make_async_copy(tb[u].at[pl.ds(0, ROW_W)],
                                  out_hbm.at[0], sw[u]).wait()

    for t in range(DEPTH - 1):
        start_gather(t, t % DEPTH)

    def step_body(q, carry):
        for u in range(DEPTH):
            t = DEPTH * q + u

            @pl.when(t < CH_PER_W)
            def _():
                @pl.when(t >= DEPTH)
                def _():
                    wait_write(u)
                wait_gather(t, u)
                _transpose_chunk(rows[u], tb[u])
                start_write(t, u)

            @pl.when(t + DEPTH - 1 < CH_PER_W)
            def _():
                start_gather(t + DEPTH - 1, (u + DEPTH - 1) % DEPTH)
        return carry

    lax.fori_loop(0, (CH_PER_W + DEPTH - 1) // DEPTH, step_body, 0)
    for u in range(DEPTH):
        wait_write(u)


@jax.jit
def _sc_gather(ids_flat, weight):
    mesh = plsc.VectorSubcoreMesh(core_axis_name="c", subcore_axis_name="s")

    def body(ids_hbm, table_hbm, out_hbm,
             idx_v, r0, r1, r2, r3, t0, t1, t2, t3,
             g0, g1, g2, g3, w0, w1, w2, w3):
        _gather_body(ids_hbm, table_hbm, out_hbm, idx_v,
                     [r0, r1, r2, r3], [t0, t1, t2, t3],
                     [g0, g1, g2, g3], [w0, w1, w2, w3])

    return pl.kernel(
        body,
        out_type=jax.ShapeDtypeStruct(
            (SEQ * (EMBED_DIM // 8) * J_PER_S, ROW_W), jnp.float32),
        mesh=mesh,
        scratch_types=(
            [pltpu.VMEM((IDX_PER_W,), jnp.int32)]
            + [pltpu.VMEM((NBLK, EMBED_DIM), jnp.float32)] * DEPTH
            + [pltpu.VMEM((EMBED_DIM * NBLK,), jnp.float32)] * DEPTH
            + [pltpu.SemaphoreType.DMA] * (2 * DEPTH)
        ),
        compiler_params=pltpu.CompilerParams(
            use_tc_tiling_on_sc=False, needs_layout_passes=False),
    )(ids_flat, weight)


def kernel(input_ids, weight):
    ids_flat = input_ids.astype(jnp.int32).T.reshape(-1)
    out2 = _sc_gather(ids_flat, weight)
    out5 = out2.reshape(SEQ, EMBED_DIM // 8, J_PER_S, 8, NBLK)
    return jnp.transpose(out5, (2, 4, 0, 1, 3)).reshape(BATCH, SEQ, EMBED_DIM)


# speedup vs baseline: 1.1131x; 1.0013x over previous
import functools

import jax
import jax.numpy as jnp
from jax import lax
from jax.experimental import pallas as pl
from jax.experimental.pallas import tpu as pltpu
from jax.experimental.pallas import tpu_sc as plsc

VOCAB = 1000000
EMBED_DIM = 32
SEQ = 50
BATCH = 16384

NC = 2
NS = 16
NW = NC * NS

LANE = 16
NBLK = 128
N_CHUNKS_TOTAL = SEQ * (BATCH // NBLK)
CH_PER_W = N_CHUNKS_TOTAL // NW
IDX_PER_W = CH_PER_W * NBLK
J_PER_S = BATCH // NBLK
ROW_W = 8 * NBLK
DEPTH = 4


def _transpose_chunk(rows, tbuf):
    iota = lax.iota(jnp.int32, LANE)
    idx_r = [iota * NBLK + r for r in range(8)]
    span = (LANE - 1) * NBLK + 8
    for l in range(NBLK):
        base = l & ~7
        for d0 in range(0, EMBED_DIM, LANE):
            v = rows[l, pl.ds(d0, LANE)]
            plsc.store_scatter(
                tbuf.at[pl.ds(d0 * NBLK + base, span)], [idx_r[l & 7]], v)


def _gather_body(ids_hbm, table_hbm, out_hbm, idx_v, rows, tb, sg, sw):
    wid = lax.axis_index("s") * NC + lax.axis_index("c")
    c0 = wid * CH_PER_W

    pltpu.sync_copy(ids_hbm.at[pl.ds(c0 * NBLK, IDX_PER_W)], idx_v)

    def idx_slice(t):
        return idx_v.at[pl.ds(t * NBLK, NBLK)]

    def start_gather(t, u):
        pltpu.async_copy(table_hbm.at[idx_slice(t)], rows[u], sg[u])

    def wait_gather(t, u):
        pltpu.make_async_copy(table_hbm.at[idx_slice(t)], rows[u], sg[u]).wait()

    def start_write(t, u):
        c = c0 + t
        s = c // J_PER_S
        j = lax.rem(c, J_PER_S)
        for i in range(EMBED_DIM // 8):
            pltpu.async_copy(tb[u].at[pl.ds(i * ROW_W, ROW_W)],
                             out_hbm.at[(s * 4 + i) * J_PER_S + j], sw[u])

    def wait_write(u):
        for _ in range(EMBED_DIM // 8):
            pltpu.make_async_copy(tb[u].at[pl.ds(0, ROW_W)],
                                  out_hbm.at[0], sw[u]).wait()

    for t in range(DEPTH - 1):
        start_gather(t, t % DEPTH)

    def step_body(q, carry):
        for u in range(DEPTH):
            t = DEPTH * q + u

            @pl.when(t < CH_PER_W)
            def _():
                @pl.when(t >= DEPTH)
                def _():
                    wait_write(u)
                wait_gather(t, u)
                _transpose_chunk(rows[u], tb[u])
                start_write(t, u)

            @pl.when(t + DEPTH - 1 < CH_PER_W)
            def _():
                start_gather(t + DEPTH - 1, (u + DEPTH - 1) % DEPTH)
        return carry

    lax.fori_loop(0, (CH_PER_W + DEPTH - 1) // DEPTH, step_body, 0)
    for u in range(DEPTH):
        wait_write(u)


@jax.jit
def _sc_gather(ids_flat, weight):
    mesh = plsc.VectorSubcoreMesh(core_axis_name="c", subcore_axis_name="s")

    def body(ids_hbm, table_hbm, out_hbm,
             idx_v, r0, r1, r2, r3, t0, t1, t2, t3,
             g0, g1, g2, g3, w0, w1, w2, w3):
        _gather_body(ids_hbm, table_hbm, out_hbm, idx_v,
                     [r0, r1, r2, r3], [t0, t1, t2, t3],
                     [g0, g1, g2, g3], [w0, w1, w2, w3])

    return pl.kernel(
        body,
        out_type=jax.ShapeDtypeStruct(
            (SEQ * (EMBED_DIM // 8) * J_PER_S, ROW_W), jnp.float32),
        mesh=mesh,
        scratch_types=(
            [pltpu.VMEM((IDX_PER_W,), jnp.int32)]
            + [pltpu.VMEM((NBLK, EMBED_DIM), jnp.float32)] * DEPTH
            + [pltpu.VMEM((EMBED_DIM * NBLK,), jnp.float32)] * DEPTH
            + [pltpu.SemaphoreType.DMA] * (2 * DEPTH)
        ),
        compiler_params=pltpu.CompilerParams(
            use_tc_tiling_on_sc=False, needs_layout_passes=False),
    )(ids_flat, weight)


def kernel(input_ids, weight):
    ids_flat = input_ids.astype(jnp.int32).T.reshape(-1)
    out2 = _sc_gather(ids_flat, weight)
    out5 = out2.reshape(SEQ, EMBED_DIM // 8, J_PER_S, 8, NBLK)
    return jnp.transpose(out5, (2, 4, 0, 1, 3)).reshape(BATCH, SEQ, EMBED_DIM)


# speedup vs baseline: 1.3087x; 1.1757x over previous
import functools

import jax
import jax.numpy as jnp
from jax import lax
from jax.experimental import pallas as pl
from jax.experimental.pallas import tpu as pltpu
from jax.experimental.pallas import tpu_sc as plsc

VOCAB = 1000000
EMBED_DIM = 32
SEQ = 50
BATCH = 16384

NC = 2
NS = 16
NW = NC * NS

LANE = 16
NBLK = 128
N_CHUNKS_TOTAL = SEQ * (BATCH // NBLK)
CH_PER_W = N_CHUNKS_TOTAL // NW
IDX_PER_W = CH_PER_W * NBLK
J_PER_S = BATCH // NBLK
ROW_W = 8 * NBLK
DEPTH = 4


def _transpose_chunk(rows, tbuf):
    iota128 = lax.iota(jnp.int32, LANE) * NBLK

    def body(l):
        for d0 in range(0, EMBED_DIM, LANE):
            v = rows[l, pl.ds(d0, LANE)]
            plsc.store_scatter(tbuf, [iota128 + (d0 * NBLK + l)], v)

    plsc.parallel_loop(0, NBLK, 1, unroll=8)(body)


def _gather_body(ids_hbm, table_hbm, out_hbm, idx_v, rows, tb, sg, sw):
    wid = lax.axis_index("s") * NC + lax.axis_index("c")
    c0 = wid * CH_PER_W

    pltpu.sync_copy(ids_hbm.at[pl.ds(c0 * NBLK, IDX_PER_W)], idx_v)

    def idx_slice(t):
        return idx_v.at[pl.ds(t * NBLK, NBLK)]

    def start_gather(t, u):
        pltpu.async_copy(table_hbm.at[idx_slice(t)], rows[u], sg[u])

    def wait_gather(t, u):
        pltpu.make_async_copy(table_hbm.at[idx_slice(t)], rows[u], sg[u]).wait()

    def start_write(t, u):
        c = c0 + t
        s = c // J_PER_S
        j = lax.rem(c, J_PER_S)
        for i in range(EMBED_DIM // 8):
            pltpu.async_copy(tb[u].at[pl.ds(i * ROW_W, ROW_W)],
                             out_hbm.at[(s * 4 + i) * J_PER_S + j], sw[u])

    def wait_write(u):
        for _ in range(EMBED_DIM // 8):
            pltpu.make_async_copy(tb[u].at[pl.ds(0, ROW_W)],
                                  out_hbm.at[0], sw[u]).wait()

    for t in range(DEPTH - 1):
        start_gather(t, t % DEPTH)

    def step_body(q, carry):
        for u in range(DEPTH):
            t = DEPTH * q + u

            @pl.when(t < CH_PER_W)
            def _():
                @pl.when(t >= DEPTH)
                def _():
                    wait_write(u)
                wait_gather(t, u)
                _transpose_chunk(rows[u], tb[u])
                start_write(t, u)

            @pl.when(t + DEPTH - 1 < CH_PER_W)
            def _():
                start_gather(t + DEPTH - 1, (u + DEPTH - 1) % DEPTH)
        return carry

    lax.fori_loop(0, (CH_PER_W + DEPTH - 1) // DEPTH, step_body, 0)
    for u in range(DEPTH):
        wait_write(u)


@jax.jit
def _sc_gather(ids_flat, weight):
    mesh = plsc.VectorSubcoreMesh(core_axis_name="c", subcore_axis_name="s")

    def body(ids_hbm, table_hbm, out_hbm,
             idx_v, r0, r1, r2, r3, t0, t1, t2, t3,
             g0, g1, g2, g3, w0, w1, w2, w3):
        _gather_body(ids_hbm, table_hbm, out_hbm, idx_v,
                     [r0, r1, r2, r3], [t0, t1, t2, t3],
                     [g0, g1, g2, g3], [w0, w1, w2, w3])

    return pl.kernel(
        body,
        out_type=jax.ShapeDtypeStruct(
            (SEQ * (EMBED_DIM // 8) * J_PER_S, ROW_W), jnp.float32),
        mesh=mesh,
        scratch_types=(
            [pltpu.VMEM((IDX_PER_W,), jnp.int32)]
            + [pltpu.VMEM((NBLK, EMBED_DIM), jnp.float32)] * DEPTH
            + [pltpu.VMEM((EMBED_DIM * NBLK,), jnp.float32)] * DEPTH
            + [pltpu.SemaphoreType.DMA] * (2 * DEPTH)
        ),
        compiler_params=pltpu.CompilerParams(
            use_tc_tiling_on_sc=False, needs_layout_passes=False),
    )(ids_flat, weight)


def kernel(input_ids, weight):
    ids_flat = input_ids.astype(jnp.int32).T.reshape(-1)
    out2 = _sc_gather(ids_flat, weight)
    out5 = out2.reshape(SEQ, EMBED_DIM // 8, J_PER_S, 8, NBLK)
    return jnp.transpose(out5, (2, 4, 0, 1, 3)).reshape(BATCH, SEQ, EMBED_DIM)


# speedup vs baseline: 1.3131x; 1.0034x over previous
import functools

import jax
import jax.numpy as jnp
from jax import lax
from jax.experimental import pallas as pl
from jax.experimental.pallas import tpu as pltpu
from jax.experimental.pallas import tpu_sc as plsc

VOCAB = 1000000
EMBED_DIM = 32
SEQ = 50
BATCH = 16384

NC = 2
NS = 16
NW = NC * NS

LANE = 16
NBLK = 128
N_CHUNKS_TOTAL = SEQ * (BATCH // NBLK)
CH_PER_W = N_CHUNKS_TOTAL // NW
IDX_PER_W = CH_PER_W * NBLK
J_PER_S = BATCH // NBLK
ROW_W = 8 * NBLK
DEPTH = 4


def _transpose_chunk(rows, tbuf):
    iota128 = lax.iota(jnp.int32, LANE) * NBLK

    def body(l):
        for d0 in range(0, EMBED_DIM, LANE):
            v = rows[l, pl.ds(d0, LANE)]
            plsc.store_scatter(tbuf, [iota128 + (d0 * NBLK + l)], v)

    plsc.parallel_loop(0, NBLK, 1, unroll=16)(body)


def _gather_body(ids_hbm, table_hbm, out_hbm, idx_v, rows, tb, sg, sw):
    wid = lax.axis_index("s") * NC + lax.axis_index("c")
    c0 = wid * CH_PER_W

    pltpu.sync_copy(ids_hbm.at[pl.ds(c0 * NBLK, IDX_PER_W)], idx_v)

    def idx_slice(t):
        return idx_v.at[pl.ds(t * NBLK, NBLK)]

    def start_gather(t, u):
        pltpu.async_copy(table_hbm.at[idx_slice(t)], rows[u], sg[u])

    def wait_gather(t, u):
        pltpu.make_async_copy(table_hbm.at[idx_slice(t)], rows[u], sg[u]).wait()

    def start_write(t, u):
        c = c0 + t
        s = c // J_PER_S
        j = lax.rem(c, J_PER_S)
        for i in range(EMBED_DIM // 8):
            pltpu.async_copy(tb[u].at[pl.ds(i * ROW_W, ROW_W)],
                             out_hbm.at[(s * 4 + i) * J_PER_S + j], sw[u])

    def wait_write(u):
        for _ in range(EMBED_DIM // 8):
            pltpu.make_async_copy(tb[u].at[pl.ds(0, ROW_W)],
                                  out_hbm.at[0], sw[u]).wait()

    for t in range(DEPTH - 1):
        start_gather(t, t % DEPTH)

    def step_body(q, carry):
        for u in range(DEPTH):
            t = DEPTH * q + u

            @pl.when(t < CH_PER_W)
            def _():
                @pl.when(t >= DEPTH)
                def _():
                    wait_write(u)
                wait_gather(t, u)
                _transpose_chunk(rows[u], tb[u])
                start_write(t, u)

            @pl.when(t + DEPTH - 1 < CH_PER_W)
            def _():
                start_gather(t + DEPTH - 1, (u + DEPTH - 1) % DEPTH)
        return carry

    lax.fori_loop(0, (CH_PER_W + DEPTH - 1) // DEPTH, step_body, 0)
    for u in range(DEPTH):
        wait_write(u)


@jax.jit
def _sc_gather(ids_flat, weight):
    mesh = plsc.VectorSubcoreMesh(core_axis_name="c", subcore_axis_name="s")

    def body(ids_hbm, table_hbm, out_hbm,
             idx_v, r0, r1, r2, r3, t0, t1, t2, t3,
             g0, g1, g2, g3, w0, w1, w2, w3):
        _gather_body(ids_hbm, table_hbm, out_hbm, idx_v,
                     [r0, r1, r2, r3], [t0, t1, t2, t3],
                     [g0, g1, g2, g3], [w0, w1, w2, w3])

    return pl.kernel(
        body,
        out_type=jax.ShapeDtypeStruct(
            (SEQ * (EMBED_DIM // 8) * J_PER_S, ROW_W), jnp.float32),
        mesh=mesh,
        scratch_types=(
            [pltpu.VMEM((IDX_PER_W,), jnp.int32)]
            + [pltpu.VMEM((NBLK, EMBED_DIM), jnp.float32)] * DEPTH
            + [pltpu.VMEM((EMBED_DIM * NBLK,), jnp.float32)] * DEPTH
            + [pltpu.SemaphoreType.DMA] * (2 * DEPTH)
        ),
        compiler_params=pltpu.CompilerParams(
            use_tc_tiling_on_sc=False, needs_layout_passes=False),
    )(ids_flat, weight)


def kernel(input_ids, weight):
    ids_flat = input_ids.astype(jnp.int32).T.reshape(-1)
    out2 = _sc_gather(ids_flat, weight)
    out5 = out2.reshape(SEQ, EMBED_DIM // 8, J_PER_S, 8, NBLK)
    return jnp.transpose(out5, (2, 4, 0, 1, 3)).reshape(BATCH, SEQ, EMBED_DIM)


# speedup vs baseline: 1.8934x; 1.4419x over previous
import functools

import jax
import jax.numpy as jnp
from jax import lax
from jax.experimental import pallas as pl
from jax.experimental.pallas import tpu as pltpu
from jax.experimental.pallas import tpu_sc as plsc

VOCAB = 1000000
EMBED_DIM = 32
SEQ = 50
BATCH = 16384

NC = 2
NS = 16
NW = NC * NS

LANE = 16
NBLK = 128
N_CHUNKS_TOTAL = SEQ * (BATCH // NBLK)
CH_PER_W = N_CHUNKS_TOTAL // NW
IDX_PER_W = CH_PER_W * NBLK
J_PER_S = BATCH // NBLK
ROW_W = 8 * NBLK
TPITCH = NBLK + 1
DEPTH = 4


def _transpose_chunk(rows, tbuf):
    iota = lax.iota(jnp.int32, LANE)

    def body(l):
        lvec = jnp.full((LANE,), 0, jnp.int32) + l
        for d0 in range(0, EMBED_DIM, LANE):
            v = rows[l, pl.ds(d0, LANE)]
            plsc.store_scatter(tbuf, [iota + d0, lvec], v)

    plsc.parallel_loop(0, NBLK, 1, unroll=16)(body)


def _gather_body(ids_hbm, table_hbm, out_hbm, idx_v, rows, tb, sg, sw):
    wid = lax.axis_index("s") * NC + lax.axis_index("c")
    c0 = wid * CH_PER_W

    pltpu.sync_copy(ids_hbm.at[pl.ds(c0 * NBLK, IDX_PER_W)], idx_v)

    def idx_slice(t):
        return idx_v.at[pl.ds(t * NBLK, NBLK)]

    def start_gather(t, u):
        pltpu.async_copy(table_hbm.at[idx_slice(t)], rows[u], sg[u])

    def wait_gather(t, u):
        pltpu.make_async_copy(table_hbm.at[idx_slice(t)], rows[u], sg[u]).wait()

    def start_write(t, u):
        c = c0 + t
        s = c // J_PER_S
        j = lax.rem(c, J_PER_S)
        for i in range(EMBED_DIM // 8):
            pltpu.async_copy(tb[u].at[pl.ds(i * 8, 8), pl.ds(0, NBLK)],
                             out_hbm.at[(s * 4 + i) * J_PER_S + j], sw[u])

    def wait_write(u):
        for _ in range(EMBED_DIM // 8):
            pltpu.make_async_copy(tb[u].at[pl.ds(0, 8), pl.ds(0, NBLK)],
                                  out_hbm.at[0], sw[u]).wait()

    for t in range(DEPTH - 1):
        start_gather(t, t % DEPTH)

    def step_body(q, carry):
        for u in range(DEPTH):
            t = DEPTH * q + u

            @pl.when(t < CH_PER_W)
            def _():
                @pl.when(t >= DEPTH)
                def _():
                    wait_write(u)
                wait_gather(t, u)
                _transpose_chunk(rows[u], tb[u])
                start_write(t, u)

            @pl.when(t + DEPTH - 1 < CH_PER_W)
            def _():
                start_gather(t + DEPTH - 1, (u + DEPTH - 1) % DEPTH)
        return carry

    lax.fori_loop(0, (CH_PER_W + DEPTH - 1) // DEPTH, step_body, 0)
    for u in range(DEPTH):
        wait_write(u)


@jax.jit
def _sc_gather(ids_flat, weight):
    mesh = plsc.VectorSubcoreMesh(core_axis_name="c", subcore_axis_name="s")

    def body(ids_hbm, table_hbm, out_hbm,
             idx_v, r0, r1, r2, r3, t0, t1, t2, t3,
             g0, g1, g2, g3, w0, w1, w2, w3):
        _gather_body(ids_hbm, table_hbm, out_hbm, idx_v,
                     [r0, r1, r2, r3], [t0, t1, t2, t3],
                     [g0, g1, g2, g3], [w0, w1, w2, w3])

    return pl.kernel(
        body,
        out_type=jax.ShapeDtypeStruct(
            (SEQ * (EMBED_DIM // 8) * J_PER_S, 8, NBLK), jnp.float32),
        mesh=mesh,
        scratch_types=(
            [pltpu.VMEM((IDX_PER_W,), jnp.int32)]
            + [pltpu.VMEM((NBLK, EMBED_DIM), jnp.float32)] * DEPTH
            + [pltpu.VMEM((EMBED_DIM, TPITCH), jnp.float32)] * DEPTH
            + [pltpu.SemaphoreType.DMA] * (2 * DEPTH)
        ),
        compiler_params=pltpu.CompilerParams(
            use_tc_tiling_on_sc=False, needs_layout_passes=False),
    )(ids_flat, weight)


def kernel(input_ids, weight):
    ids_flat = input_ids.astype(jnp.int32).T.reshape(-1)
    out3 = _sc_gather(ids_flat, weight)
    out5 = out3.reshape(SEQ, EMBED_DIM // 8, J_PER_S, 8, NBLK)
    return jnp.transpose(out5, (2, 4, 0, 1, 3)).reshape(BATCH, SEQ, EMBED_DIM)
